# Initial kernel scaffold; baseline (speedup 1.0000x reference)
import jax
import jax.numpy as jnp
from jax.experimental import pallas as pl


def kernel(x, Wq, bq, Wk, bk, Wv, bv, Wo, bo, b_start_loc, b_seq_len, max_seq_len):
    raise NotImplementedError("write your pallas kernel here")


# speedup vs baseline: 1.5296x; 1.5296x over previous
import functools

import jax
import jax.numpy as jnp
import numpy as np
from jax.experimental import pallas as pl
from jax.experimental.pallas import tpu as pltpu

B = 8
S = 1024
H = 2048
HQ = 16
HKV = 4
D = 128
G = HQ // HKV
NQ = HQ * D
NKV = HKV * D

BM = 512
BQ = 256


def _qkv_kernel(x_ref, w_ref, b_ref, q_ref, k_ref, v_ref):
    x = x_ref[...].astype(jnp.bfloat16)
    y = jnp.dot(x, w_ref[...], preferred_element_type=jnp.float32) + b_ref[...]
    y = y.astype(jnp.bfloat16)
    q_ref[...] = y[:, :NQ]
    k_ref[...] = y[:, NQ:NQ + NKV]
    v_ref[...] = y[:, NQ + NKV:]


def _attn_kernel(q_ref, k_ref, v_ref, o_ref):
    scale = jnp.float32(1.0 / np.sqrt(D))
    i = pl.program_id(2)
    k = k_ref[...]
    v = v_ref[...]
    row = jax.lax.broadcasted_iota(jnp.int32, (BQ, S), 0) + i * BQ
    col = jax.lax.broadcasted_iota(jnp.int32, (BQ, S), 1)
    mask = row >= col
    for g in range(G):
        q_g = q_ref[:, g * D:(g + 1) * D]
        s = jax.lax.dot_general(q_g, k, (((1,), (1,)), ((), ())),
                                preferred_element_type=jnp.float32)
        s = s * scale
        s = jnp.where(mask, s, jnp.float32(-1e30))
        m = jnp.max(s, axis=-1, keepdims=True)
        e = jnp.exp(s - m)
        denom = jnp.sum(e, axis=-1, keepdims=True)
        p = (e / denom).astype(jnp.bfloat16)
        o_g = jnp.dot(p, v, preferred_element_type=jnp.float32)
        o_ref[:, g * D:(g + 1) * D] = o_g.astype(jnp.bfloat16)


def _out_kernel(x_ref, w_ref, b_ref, o_ref):
    o_ref[...] = (jnp.dot(x_ref[...], w_ref[...],
                          preferred_element_type=jnp.float32) + b_ref[...])


def kernel(x, Wq, bq, Wk, bk, Wv, bv, Wo, bo, b_start_loc, b_seq_len, max_seq_len):
    T = x.shape[0]
    Wqkv = jnp.concatenate([Wq, Wk, Wv], axis=1).astype(jnp.bfloat16)
    bqkv = jnp.concatenate([bq, bk, bv])[None, :]
    Nqkv = NQ + 2 * NKV

    q, k, v = pl.pallas_call(
        _qkv_kernel,
        grid=(T // BM,),
        in_specs=[
            pl.BlockSpec((BM, H), lambda i: (i, 0)),
            pl.BlockSpec((H, Nqkv), lambda i: (0, 0)),
            pl.BlockSpec((1, Nqkv), lambda i: (0, 0)),
        ],
        out_specs=[
            pl.BlockSpec((BM, NQ), lambda i: (i, 0)),
            pl.BlockSpec((BM, NKV), lambda i: (i, 0)),
            pl.BlockSpec((BM, NKV), lambda i: (i, 0)),
        ],
        out_shape=[
            jax.ShapeDtypeStruct((T, NQ), jnp.bfloat16),
            jax.ShapeDtypeStruct((T, NKV), jnp.bfloat16),
            jax.ShapeDtypeStruct((T, NKV), jnp.bfloat16),
        ],
        compiler_params=pltpu.CompilerParams(
            dimension_semantics=("parallel",),
            vmem_limit_bytes=100 * 1024 * 1024,
        ),
    )(x, Wqkv, bqkv)

    o = pl.pallas_call(
        _attn_kernel,
        grid=(B, HKV, S // BQ),
        in_specs=[
            pl.BlockSpec((BQ, G * D), lambda b, h, i: (b * (S // BQ) + i, h)),
            pl.BlockSpec((S, D), lambda b, h, i: (b, h)),
            pl.BlockSpec((S, D), lambda b, h, i: (b, h)),
        ],
        out_specs=pl.BlockSpec((BQ, G * D), lambda b, h, i: (b * (S // BQ) + i, h)),
        out_shape=jax.ShapeDtypeStruct((T, NQ), jnp.bfloat16),
        compiler_params=pltpu.CompilerParams(
            dimension_semantics=("parallel", "arbitrary", "arbitrary"),
            vmem_limit_bytes=100 * 1024 * 1024,
        ),
    )(q, k, v)

    out = pl.pallas_call(
        _out_kernel,
        grid=(T // BM,),
        in_specs=[
            pl.BlockSpec((BM, NQ), lambda i: (i, 0)),
            pl.BlockSpec((NQ, H), lambda i: (0, 0)),
            pl.BlockSpec((1, H), lambda i: (0, 0)),
        ],
        out_specs=pl.BlockSpec((BM, H), lambda i: (i, 0)),
        out_shape=jax.ShapeDtypeStruct((T, H), jnp.float32),
        compiler_params=pltpu.CompilerParams(
            dimension_semantics=("parallel",),
            vmem_limit_bytes=100 * 1024 * 1024,
        ),
    )(o, Wo.astype(jnp.bfloat16), bo[None, :])

    return out


# speedup vs baseline: 2.0573x; 1.3450x over previous
import jax
import jax.numpy as jnp
import numpy as np
from jax.experimental import pallas as pl
from jax.experimental.pallas import tpu as pltpu

B = 8
S = 1024
H = 2048
HQ = 16
HKV = 4
D = 128
G = HQ // HKV
NQ = HQ * D
NKV = HKV * D

BM = 512
BQ = 256
NCHUNK = S // BQ


def _qkv_kernel(x_ref, w_ref, b_ref, q_ref, k_ref, v_ref):
    x = x_ref[...].astype(jnp.bfloat16)
    y = jnp.dot(x, w_ref[...], preferred_element_type=jnp.float32) + b_ref[...]
    y = y.astype(jnp.bfloat16)
    q_ref[...] = y[:, :NQ]
    k_ref[...] = y[:, NQ:NQ + NKV]
    v_ref[...] = y[:, NQ + NKV:]


def _attn_kernel(q_ref, k_ref, v_ref, o_ref):
    k = k_ref[...]
    v = v_ref[...]
    colf = jax.lax.broadcasted_iota(jnp.int32, (BQ, S), 1)
    rowf = jax.lax.broadcasted_iota(jnp.int32, (BQ, 1), 0)
    for g in range(G):
        for i in range(NCHUNK):
            ext = (i + 1) * BQ
            q_g = q_ref[i * BQ:(i + 1) * BQ, g * D:(g + 1) * D]
            s = jax.lax.dot_general(q_g, k[:ext, :], (((1,), (1,)), ((), ())),
                                    preferred_element_type=jnp.float32)
            mask = (rowf + i * BQ) >= colf[:, :ext]
            s = jnp.where(mask, s, jnp.float32(-1e30))
            m = jnp.max(s, axis=-1, keepdims=True)
            e = jnp.exp(s - m)
            denom = jnp.sum(e, axis=-1, keepdims=True)
            o_g = jnp.dot(e.astype(jnp.bfloat16), v[:ext, :],
                          preferred_element_type=jnp.float32)
            o_g = o_g * (1.0 / denom)
            o_ref[i * BQ:(i + 1) * BQ, g * D:(g + 1) * D] = o_g.astype(jnp.bfloat16)


def _out_kernel(x_ref, w_ref, b_ref, o_ref):
    o_ref[...] = (jnp.dot(x_ref[...], w_ref[...],
                          preferred_element_type=jnp.float32) + b_ref[...])


def kernel(x, Wq, bq, Wk, bk, Wv, bv, Wo, bo, b_start_loc, b_seq_len, max_seq_len):
    T = x.shape[0]
    scale = 1.0 / np.sqrt(D)
    Wqkv = jnp.concatenate([Wq * scale, Wk, Wv], axis=1).astype(jnp.bfloat16)
    bqkv = jnp.concatenate([bq * scale, bk, bv])[None, :]
    Nqkv = NQ + 2 * NKV

    q, k, v = pl.pallas_call(
        _qkv_kernel,
        grid=(T // BM,),
        in_specs=[
            pl.BlockSpec((BM, H), lambda i: (i, 0)),
            pl.BlockSpec((H, Nqkv), lambda i: (0, 0)),
            pl.BlockSpec((1, Nqkv), lambda i: (0, 0)),
        ],
        out_specs=[
            pl.BlockSpec((BM, NQ), lambda i: (i, 0)),
            pl.BlockSpec((BM, NKV), lambda i: (i, 0)),
            pl.BlockSpec((BM, NKV), lambda i: (i, 0)),
        ],
        out_shape=[
            jax.ShapeDtypeStruct((T, NQ), jnp.bfloat16),
            jax.ShapeDtypeStruct((T, NKV), jnp.bfloat16),
            jax.ShapeDtypeStruct((T, NKV), jnp.bfloat16),
        ],
        compiler_params=pltpu.CompilerParams(
            dimension_semantics=(pltpu.PARALLEL,),
            vmem_limit_bytes=100 * 1024 * 1024,
        ),
    )(x, Wqkv, bqkv)

    o = pl.pallas_call(
        _attn_kernel,
        grid=(B, HKV),
        in_specs=[
            pl.BlockSpec((S, G * D), lambda b, h: (b, h)),
            pl.BlockSpec((S, D), lambda b, h: (b, h)),
            pl.BlockSpec((S, D), lambda b, h: (b, h)),
        ],
        out_specs=pl.BlockSpec((S, G * D), lambda b, h: (b, h)),
        out_shape=jax.ShapeDtypeStruct((T, NQ), jnp.bfloat16),
        compiler_params=pltpu.CompilerParams(
            dimension_semantics=(pltpu.PARALLEL, pltpu.ARBITRARY),
            vmem_limit_bytes=100 * 1024 * 1024,
        ),
    )(q, k, v)

    out = pl.pallas_call(
        _out_kernel,
        grid=(T // BM,),
        in_specs=[
            pl.BlockSpec((BM, NQ), lambda i: (i, 0)),
            pl.BlockSpec((NQ, H), lambda i: (0, 0)),
            pl.BlockSpec((1, H), lambda i: (0, 0)),
        ],
        out_specs=pl.BlockSpec((BM, H), lambda i: (i, 0)),
        out_shape=jax.ShapeDtypeStruct((T, H), jnp.float32),
        compiler_params=pltpu.CompilerParams(
            dimension_semantics=(pltpu.PARALLEL,),
            vmem_limit_bytes=100 * 1024 * 1024,
        ),
    )(o, Wo.astype(jnp.bfloat16), bo[None, :])

    return out
